# Initial kernel scaffold; baseline (speedup 1.0000x reference)
import jax
import jax.numpy as jnp
from jax.experimental import pallas as pl


def kernel(logits):
    raise NotImplementedError("write your pallas kernel here")


# speedup vs baseline: 3.9581x; 3.9581x over previous
import functools

import jax
import jax.numpy as jnp
from jax import lax
from jax.experimental import pallas as pl
from jax.experimental.pallas import tpu as pltpu
from jax.experimental.pallas import tpu_sc as plsc

B = 128
N = 4096
K = 32
TAU_INV = 2.0
EPS = 1e-7
L = 16
NC, NS = 2, 16
NW = NC * NS
RPW = B // NW
CHUNKS = N // L
UNROLL = 8


def _lane_allreduce(v, op):
    dnums = lax.GatherDimensionNumbers(
        offset_dims=(), collapsed_slice_dims=(0,), start_index_map=(0,))
    for shift in (8, 4, 2, 1):
        idx = (lax.iota(jnp.int32, L) + shift) & (L - 1)
        perm = lax.gather(v, idx[:, None], dnums, slice_sizes=(1,),
                          mode=lax.GatherScatterMode.PROMISE_IN_BOUNDS)
        v = op(v, perm)
    return v


def _sc_body(l_hbm, g_hbm, out_hbm, wbuf, sbuf, khot):
    cid = lax.axis_index("c")
    sid = lax.axis_index("s")
    wid = sid * NC + cid
    base = wid * RPW

    pltpu.sync_copy(l_hbm.at[pl.ds(base, RPW)], wbuf)
    pltpu.sync_copy(g_hbm.at[pl.ds(base, RPW)], sbuf)

    for r in range(RPW):
        def maxbody(i, acc, r=r):
            for u in range(UNROLL):
                ix = pl.ds(i * (L * UNROLL) + u * L, L)
                w = wbuf[r, ix] + sbuf[r, ix]
                wbuf[r, ix] = w
                acc = jnp.maximum(acc, w)
            return acc

        acc0 = jnp.full((L,), -jnp.inf, dtype=jnp.float32)
        rmax = _lane_allreduce(lax.fori_loop(0, CHUNKS // UNROLL, maxbody, acc0),
                               jnp.maximum)

        def initbody(i, acc, r=r, rmax=rmax):
            for u in range(UNROLL):
                ix = pl.ds(i * (L * UNROLL) + u * L, L)
                sv = jnp.exp((wbuf[r, ix] - rmax) * TAU_INV)
                sbuf[r, ix] = sv
                khot[r, ix] = jnp.zeros((L,), dtype=jnp.float32)
                acc = acc + sv
            return acc

        acc0 = jnp.zeros((L,), dtype=jnp.float32)
        denom0 = _lane_allreduce(lax.fori_loop(0, CHUNKS // UNROLL, initbody, acc0),
                                 jnp.add)

        def roundbody(t, denom, r=r):
            inv = 1.0 / denom

            def chunkbody(i, acc):
                for u in range(UNROLL):
                    ix = pl.ds(i * (L * UNROLL) + u * L, L)
                    p = sbuf[r, ix] * inv
                    khot[r, ix] = khot[r, ix] + p
                    m = jnp.maximum(1.0 - p, EPS)
                    sn = p * (m * m)
                    sbuf[r, ix] = sn
                    acc = acc + sn
                return acc

            acc = lax.fori_loop(0, CHUNKS // UNROLL, chunkbody,
                                jnp.zeros((L,), dtype=jnp.float32))
            return _lane_allreduce(acc, jnp.add)

        lax.fori_loop(0, K, roundbody, denom0)

    pltpu.sync_copy(khot, out_hbm.at[pl.ds(base, RPW)])


@functools.partial(
    pl.kernel,
    out_type=jax.ShapeDtypeStruct((B, N), jnp.float32),
    mesh=plsc.VectorSubcoreMesh(core_axis_name="c", subcore_axis_name="s"),
    scratch_types=[
        pltpu.VMEM((RPW, N), jnp.float32),
        pltpu.VMEM((RPW, N), jnp.float32),
        pltpu.VMEM((RPW, N), jnp.float32),
    ],
)
def _sample_subset_sc(l_hbm, g_hbm, out_hbm, wbuf, sbuf, khot):
    _sc_body(l_hbm, g_hbm, out_hbm, wbuf, sbuf, khot)


def kernel(logits):
    l = jnp.squeeze(logits, 2)
    u = jax.random.uniform(jax.random.key(42), l.shape,
                           minval=1e-20, maxval=1.0, dtype=l.dtype)
    g = -jnp.log(-jnp.log(u))
    out = _sample_subset_sc(l, g)
    return jnp.expand_dims(out, -1)
